# baseline (speedup 1.0000x reference)
import jax
import jax.numpy as jnp
from jax.experimental import pallas as pl

S = 2048
B = 2
H = 768
NH = 12
HPH = 64
TQ = 256
NT = S // TQ
SCALE = 1.0 / (HPH ** 0.5)


def _qkv_kernel(x_ref, w_ref, b_ref, o_ref):
    o_ref[...] = (
        jnp.dot(x_ref[...], w_ref[...], preferred_element_type=jnp.float32)
        + b_ref[...]
    ).astype(jnp.bfloat16)


def _rep_dense(rep, wd_ref, bd_ref):
    return (
        jnp.dot(rep.astype(jnp.bfloat16), wd_ref[...],
                preferred_element_type=jnp.float32)
        + bd_ref[...]
    )


def _attn_kernel(kv_ref, mask_ref, wd_ref, bd_ref, o_ref):
    i = pl.program_id(1)
    q_all = kv_ref[pl.ds(i * TQ, TQ), :]
    maskb = mask_ref[0] != 0
    outs = []
    for h in range(NH):
        base = h * 3 * HPH
        q = q_all[:, base:base + HPH] * jnp.bfloat16(SCALE)
        k = kv_ref[:, base + HPH:base + 2 * HPH]
        v = kv_ref[:, base + 2 * HPH:base + 3 * HPH]
        s = jax.lax.dot_general(
            q, k, (((1,), (1,)), ((), ())), preferred_element_type=jnp.float32
        )
        p = jnp.where(maskb, jnp.exp(s), 0.0)
        l = jnp.sum(p, axis=1, keepdims=True)
        o = jax.lax.dot_general(
            p.astype(jnp.bfloat16), v, (((1,), (0,)), ((), ())),
            preferred_element_type=jnp.float32,
        )
        outs.append(o / jnp.where(l == 0.0, 1.0, l))
    rep = jnp.concatenate(outs, axis=1)
    o_ref[...] = _rep_dense(rep, wd_ref, bd_ref)


def kernel(hidden_states, attention_mask, W_qkv, b_qkv, W_dense, b_dense):
    x = jnp.transpose(hidden_states.astype(jnp.bfloat16), (1, 0, 2)).reshape(B * S, H)
    mask8 = (attention_mask.reshape(B, S, S) != 0).astype(jnp.int8)
    w_qkv16 = W_qkv.astype(jnp.bfloat16)
    w_dense16 = W_dense.astype(jnp.bfloat16)

    mixed = pl.pallas_call(
        _qkv_kernel,
        grid=(B * NT,),
        in_specs=[
            pl.BlockSpec((TQ, H), lambda i: (i, 0)),
            pl.BlockSpec((H, 3 * H), lambda i: (0, 0)),
            pl.BlockSpec((1, 3 * H), lambda i: (0, 0)),
        ],
        out_specs=pl.BlockSpec((TQ, 3 * H), lambda i: (i, 0)),
        out_shape=jax.ShapeDtypeStruct((B * S, 3 * H), jnp.bfloat16),
    )(x, w_qkv16, b_qkv.reshape(1, 3 * H))

    out2 = pl.pallas_call(
        _attn_kernel,
        grid=(B, NT),
        in_specs=[
            pl.BlockSpec((S, 3 * H), lambda b, i: (b, 0)),
            pl.BlockSpec((1, TQ, S), lambda b, i: (b, i, 0)),
            pl.BlockSpec((H, H), lambda b, i: (0, 0)),
            pl.BlockSpec((1, H), lambda b, i: (0, 0)),
        ],
        out_specs=pl.BlockSpec((TQ, H), lambda b, i: (b * NT + i, 0)),
        out_shape=jax.ShapeDtypeStruct((B * S, H), jnp.float32),
    )(mixed, mask8, w_dense16, b_dense.reshape(1, H))

    return out2.reshape(B, S, H).transpose(1, 0, 2)
